# baseline (speedup 1.0000x reference)
import functools

import jax
import jax.numpy as jnp
from jax import lax
from jax.experimental import pallas as pl
from jax.experimental.pallas import tpu as pltpu
from jax.experimental.pallas import tpu_sc as plsc

N = 10000
E = 320000
D = 128
R = 16
BN_EPS = 1e-3

NC = 2
NS = 16
NW = NC * NS
K = 40
EW = E // NW
NCHUNK = EW // K
ROWS_PS = 624
ROWS_REM_OFF = ROWS_PS * NS
ROWS_REM = N - ROWS_REM_OFF

_GRID = 10
_NB = N // _GRID
_EB = (E // D) // _GRID


def _prep_body(rc_ref, x_ref, ck_ref, g_ref, bt_ref, av_ref, rv_ref,
               xbn_ref, xb2_ref, a_ref):
    scale = g_ref[...] * (1.0 / jnp.sqrt(1.0 + BN_EPS))
    xbn = x_ref[...] * scale + bt_ref[...]
    xbn_ref[...] = xbn
    xb2_ref[...] = xbn * (ck_ref[...] + 1.0)
    rv = rv_ref[0]
    rel = jnp.zeros(rv.shape, jnp.float32)
    for r in range(R):
        rel = jnp.where(rv == r, rc_ref[r], rel)
    a_ref[0] = av_ref[0] / (1.0 + rel)


def _combine_body(p0_ref, p1_ref, w_ref, b_ref, o_ref):
    h = p0_ref[0] + p1_ref[0]
    o_ref[...] = jnp.dot(h, w_ref[...],
                         preferred_element_type=jnp.float32) + b_ref[...]


_sc_mesh = plsc.VectorSubcoreMesh(core_axis_name="c", subcore_axis_name="s")


@functools.partial(
    pl.kernel,
    out_type=jax.ShapeDtypeStruct((NC, N, D), jnp.float32),
    mesh=_sc_mesh,
    scratch_types=[
        pltpu.VMEM_SHARED((N, D), jnp.float32),
        pltpu.VMEM((K,), jnp.int32),
        pltpu.VMEM((K,), jnp.int32),
        pltpu.VMEM((K,), jnp.int32),
        pltpu.VMEM((K,), jnp.int32),
        pltpu.VMEM((K,), jnp.int32),
        pltpu.VMEM((K,), jnp.float32),
        pltpu.VMEM((K,), jnp.float32),
        pltpu.VMEM((K, D), jnp.float32),
        pltpu.VMEM((K, D), jnp.float32),
        pltpu.VMEM((K, D), jnp.float32),
        pltpu.VMEM((K, D), jnp.float32),
        pltpu.VMEM((K,), jnp.int32),
        pltpu.VMEM((K,), jnp.int32),
        pltpu.SemaphoreType.DMA,
        pltpu.SemaphoreType.DMA,
        pltpu.SemaphoreType.DMA,
        pltpu.SemaphoreType.DMA,
        pltpu.SemaphoreType.DMA,
        pltpu.SemaphoreType.DMA,
        pltpu.SemaphoreType.DMA,
    ],
)
def _sc_scatter(xbn_hbm, a_hbm, row_hbm, col_hbm, init_hbm, zero_hbm,
                out_hbm, acc, colb0, colb1, colb2, rowb0, rowb1,
                ab0, ab1, rows0, rows1, st0, st1, ix0, ix1,
                sem0, sem1, semc0, semc1, semc2, sems0, sems1):
    c = lax.axis_index("c")
    s = lax.axis_index("s")
    w = c * NS + s
    off = s * ROWS_PS

    @pl.when(c == 0)
    def _():
        pltpu.sync_copy(init_hbm.at[pl.ds(off, ROWS_PS)],
                        acc.at[pl.ds(off, ROWS_PS)])

    @pl.when(c != 0)
    def _():
        pltpu.sync_copy(zero_hbm.at[pl.ds(off, ROWS_PS)],
                        acc.at[pl.ds(off, ROWS_PS)])

    @pl.when((s == 0) & (c == 0))
    def _():
        pltpu.sync_copy(init_hbm.at[pl.ds(ROWS_REM_OFF, ROWS_REM)],
                        acc.at[pl.ds(ROWS_REM_OFF, ROWS_REM)])

    @pl.when((s == 0) & (c != 0))
    def _():
        pltpu.sync_copy(zero_hbm.at[pl.ds(ROWS_REM_OFF, ROWS_REM)],
                        acc.at[pl.ds(ROWS_REM_OFF, ROWS_REM)])

    plsc.subcore_barrier()

    ebase = w * EW
    bufs = ((rows0, rowb0, ab0, sem0),
            (rows1, rowb1, ab1, sem1))
    slots = ((st0, ix0, sems0), (st1, ix1, sems1))
    cols = ((colb0, semc0), (colb1, semc1), (colb2, semc2))

    def issue_col(i, cs):
        col_v, semc = cs
        pltpu.async_copy(col_hbm.at[pl.ds(ebase + i * K, K)], col_v, semc)

    def wait_col(i, cs):
        col_v, semc = cs
        pltpu.make_async_copy(col_hbm.at[pl.ds(ebase + i * K, K)],
                              col_v, semc).wait()

    def issue(i, buf, cs):
        rows_v, row_v, a_v, sem = buf
        pltpu.async_copy(xbn_hbm.at[cs[0]], rows_v, sem)
        pltpu.async_copy(row_hbm.at[pl.ds(ebase + i * K, K)], row_v, sem)
        pltpu.async_copy(a_hbm.at[pl.ds(ebase + i * K, K)], a_v, sem)

    def drain(i, buf, cs):
        rows_v, row_v, a_v, sem = buf
        pltpu.make_async_copy(xbn_hbm.at[cs[0]], rows_v, sem).wait()
        pltpu.make_async_copy(row_hbm.at[pl.ds(ebase + i * K, K)],
                              row_v, sem).wait()
        pltpu.make_async_copy(a_hbm.at[pl.ds(ebase + i * K, K)],
                              a_v, sem).wait()

    def scale_to(buf, slot):
        rows_v, row_v, a_v = buf[0], buf[1], buf[2]
        st, ix, _ = slot

        def group(g, carry):
            a16 = a_v[pl.ds(g * 16, 16)]
            ix[pl.ds(g * 16, 16)] = row_v[pl.ds(g * 16, 16)]
            for l in range(16):
                ae = a16[l]
                e = g * 16 + l
                for j in range(D // 16):
                    sl = pl.ds(j * 16, 16)
                    st[e, sl] = rows_v[e, sl] * ae
            return carry

        lax.fori_loop(0, K // 16, group, 0)
        ix[pl.ds(K - 16, 16)] = row_v[pl.ds(K - 16, 16)]
        a16t = a_v[pl.ds(K - 16, 16)]
        for l in range(8, 16):
            ae = a16t[l]
            e = K - 16 + l
            for j in range(D // 16):
                sl = pl.ds(j * 16, 16)
                st[e, sl] = rows_v[e, sl] * ae

    def scatter_start(slot):
        st, ix, sem = slot
        pltpu.async_copy(st, acc.at[ix], sem, add=True)

    def scatter_wait(slot):
        st, ix, sem = slot
        pltpu.make_async_copy(st, acc.at[ix], sem).wait()

    def step(j, b, v, guard_wait, do_issue, do_col):
        if do_issue:
            wait_col(j + 1, cols[(v + 1) % 3])
            issue(j + 1, bufs[1 - b], cols[(v + 1) % 3])
        drain(j, bufs[b], cols[v])
        if do_col:
            issue_col(j + 2, cols[(v + 2) % 3])
        scale_to(bufs[b], slots[b])
        if guard_wait:
            scatter_wait(slots[b])
        scatter_start(slots[b])

    issue_col(0, cols[0])
    issue_col(1, cols[1])
    wait_col(0, cols[0])
    issue(0, bufs[0], cols[0])

    step(0, 0, 0, False, True, True)
    step(1, 1, 1, False, True, True)

    def hexad(h, carry):
        j = h * 6 + 2
        step(j, 0, 2, True, True, True)
        step(j + 1, 1, 0, True, True, True)
        step(j + 2, 0, 1, True, True, True)
        step(j + 3, 1, 2, True, True, True)
        step(j + 4, 0, 0, True, True, True)
        step(j + 5, 1, 1, True, True, True)
        return carry

    lax.fori_loop(0, (NCHUNK - 10) // 6, hexad, 0)
    _tail = (
        (NCHUNK - 8, 0, 2), (NCHUNK - 7, 1, 0), (NCHUNK - 6, 0, 1),
        (NCHUNK - 5, 1, 2), (NCHUNK - 4, 0, 0), (NCHUNK - 3, 1, 1),
        (NCHUNK - 2, 0, 2), (NCHUNK - 1, 1, 0),
    )
    for j, b, v in _tail:
        step(j, b, v, True, j + 1 <= NCHUNK - 1, j + 2 <= NCHUNK - 1)
    scatter_wait(slots[0])
    scatter_wait(slots[1])

    plsc.subcore_barrier()

    pltpu.sync_copy(acc.at[pl.ds(off, ROWS_PS)],
                    out_hbm.at[c, pl.ds(off, ROWS_PS)])

    @pl.when(s == 0)
    def _():
        pltpu.sync_copy(acc.at[pl.ds(ROWS_REM_OFF, ROWS_REM)],
                        out_hbm.at[c, pl.ds(ROWS_REM_OFF, ROWS_REM)])


def kernel(x, A_vals, relation_coeffs, gamma, beta, coeff_kernel, W, b,
           edge_row, edge_col, rel_values):
    edge_row = edge_row.astype(jnp.int32)
    edge_col = edge_col.astype(jnp.int32)
    rel_values = rel_values.astype(jnp.int32)
    av2 = A_vals.reshape(_GRID, _EB, D)
    rv2 = rel_values.reshape(_GRID, _EB, D)
    g2 = gamma.reshape(1, D)
    bt2 = beta.reshape(1, D)
    b2 = b.reshape(1, D)

    xbn, xb2, a2 = pl.pallas_call(
        _prep_body,
        grid=(_GRID,),
        in_specs=[
            pl.BlockSpec(memory_space=pltpu.SMEM),
            pl.BlockSpec((_NB, D), lambda i: (i, 0)),
            pl.BlockSpec((_NB, 1), lambda i: (i, 0)),
            pl.BlockSpec((1, D), lambda i: (0, 0)),
            pl.BlockSpec((1, D), lambda i: (0, 0)),
            pl.BlockSpec((1, _EB, D), lambda i: (i, 0, 0)),
            pl.BlockSpec((1, _EB, D), lambda i: (i, 0, 0)),
        ],
        out_specs=[
            pl.BlockSpec((_NB, D), lambda i: (i, 0)),
            pl.BlockSpec((_NB, D), lambda i: (i, 0)),
            pl.BlockSpec((1, _EB, D), lambda i: (i, 0, 0)),
        ],
        out_shape=[
            jax.ShapeDtypeStruct((N, D), jnp.float32),
            jax.ShapeDtypeStruct((N, D), jnp.float32),
            jax.ShapeDtypeStruct((_GRID, _EB, D), jnp.float32),
        ],
    )(relation_coeffs, x, coeff_kernel, g2, bt2, av2, rv2)

    a_flat = a2.reshape(E)
    zeros = jnp.zeros((N, D), jnp.float32)
    partials = _sc_scatter(xbn, a_flat, edge_row, edge_col, xb2, zeros)

    out = pl.pallas_call(
        _combine_body,
        grid=(_GRID,),
        in_specs=[
            pl.BlockSpec((1, _NB, D), lambda i: (0, i, 0)),
            pl.BlockSpec((1, _NB, D), lambda i: (1, i, 0)),
            pl.BlockSpec((D, D), lambda i: (0, 0)),
            pl.BlockSpec((1, D), lambda i: (0, 0)),
        ],
        out_specs=pl.BlockSpec((_NB, D), lambda i: (i, 0)),
        out_shape=jax.ShapeDtypeStruct((N, D), jnp.float32),
    )(partials, partials, W, b2)

    return out


# speedup vs baseline: 1.3173x; 1.3173x over previous
import functools

import jax
import jax.numpy as jnp
from jax import lax
from jax.experimental import pallas as pl
from jax.experimental.pallas import tpu as pltpu
from jax.experimental.pallas import tpu_sc as plsc

N = 10000
E = 320000
D = 128
R = 16
BN_EPS = 1e-3

NC = 2
NS = 16
NW = NC * NS
EW = E // NW
K = 80
NCHUNK = EW // K
ROWS_PS = 624
ROWS_REM_OFF = ROWS_PS * NS
ROWS_REM = N - ROWS_REM_OFF

_GRID = 10
_NB = N // _GRID
_EB = (E // D) // _GRID


def _prep_body(rc_ref, x_ref, ck_ref, g_ref, bt_ref, av_ref, rv_ref,
               xbn_ref, xb2_ref, a_ref):
    scale = g_ref[...] * (1.0 / jnp.sqrt(1.0 + BN_EPS))
    xbn = x_ref[...] * scale + bt_ref[...]
    xbn_ref[...] = xbn
    xb2_ref[...] = xbn * (ck_ref[...] + 1.0)
    rv = rv_ref[0]
    rel = jnp.zeros(rv.shape, jnp.float32)
    for r in range(R):
        rel = jnp.where(rv == r, rc_ref[r], rel)
    a_ref[0] = av_ref[0] / (1.0 + rel)


def _combine_body(p0_ref, p1_ref, w_ref, b_ref, o_ref):
    h = p0_ref[0] + p1_ref[0]
    o_ref[...] = jnp.dot(h, w_ref[...],
                         preferred_element_type=jnp.float32) + b_ref[...]


_sc_mesh = plsc.VectorSubcoreMesh(core_axis_name="c", subcore_axis_name="s")


@functools.partial(
    pl.kernel,
    out_type=jax.ShapeDtypeStruct((NC, N, D), jnp.float32),
    mesh=_sc_mesh,
    scratch_types=[
        pltpu.VMEM_SHARED((N, D), jnp.float32),
        pltpu.VMEM((NCHUNK, K), jnp.int32),
        pltpu.VMEM((K,), jnp.int32),
        pltpu.VMEM((K,), jnp.int32),
        pltpu.VMEM((K,), jnp.int32),
        pltpu.VMEM((K,), jnp.float32),
        pltpu.VMEM((K,), jnp.float32),
        pltpu.VMEM((K,), jnp.float32),
        pltpu.VMEM((K, D), jnp.float32),
        pltpu.VMEM((K, D), jnp.float32),
        pltpu.VMEM((K, D), jnp.float32),
        pltpu.SemaphoreType.DMA,
        pltpu.SemaphoreType.DMA,
        pltpu.SemaphoreType.DMA,
    ],
)
def _sc_scatter(xbn_hbm, a_hbm, row_hbm, col_hbm, init_hbm, zero_hbm,
                out_hbm, acc, col2, rowb0, rowb1, rowb2, ab0, ab1, ab2,
                rows0, rows1, rows2, sem0, sem1, sem2):
    c = lax.axis_index("c")
    s = lax.axis_index("s")
    w = c * NS + s
    off = s * ROWS_PS

    @pl.when(c == 0)
    def _():
        pltpu.sync_copy(init_hbm.at[pl.ds(off, ROWS_PS)],
                        acc.at[pl.ds(off, ROWS_PS)])

    @pl.when(c != 0)
    def _():
        pltpu.sync_copy(zero_hbm.at[pl.ds(off, ROWS_PS)],
                        acc.at[pl.ds(off, ROWS_PS)])

    @pl.when((s == 0) & (c == 0))
    def _():
        pltpu.sync_copy(init_hbm.at[pl.ds(ROWS_REM_OFF, ROWS_REM)],
                        acc.at[pl.ds(ROWS_REM_OFF, ROWS_REM)])

    @pl.when((s == 0) & (c != 0))
    def _():
        pltpu.sync_copy(zero_hbm.at[pl.ds(ROWS_REM_OFF, ROWS_REM)],
                        acc.at[pl.ds(ROWS_REM_OFF, ROWS_REM)])

    pltpu.sync_copy(col_hbm.at[w], col2)

    plsc.subcore_barrier()

    ebase = w * EW
    bufs = ((rows0, rowb0, ab0, sem0),
            (rows1, rowb1, ab1, sem1),
            (rows2, rowb2, ab2, sem2))

    def issue(i, buf):
        rows_v, row_v, a_v, sem = buf
        pltpu.async_copy(xbn_hbm.at[col2.at[i]], rows_v, sem)
        pltpu.async_copy(row_hbm.at[pl.ds(ebase + i * K, K)], row_v, sem)
        pltpu.async_copy(a_hbm.at[pl.ds(ebase + i * K, K)], a_v, sem)

    def drain(i, buf):
        rows_v, row_v, a_v, sem = buf
        pltpu.make_async_copy(xbn_hbm.at[col2.at[i]], rows_v, sem).wait()
        pltpu.make_async_copy(row_hbm.at[pl.ds(ebase + i * K, K)],
                              row_v, sem).wait()
        pltpu.make_async_copy(a_hbm.at[pl.ds(ebase + i * K, K)],
                              a_v, sem).wait()

    def process(buf):
        rows_v, row_v, a_v, _ = buf

        def group(g, carry):
            a16 = a_v[pl.ds(g * 16, 16)]
            for l in range(16):
                ae = a16[l]
                e = g * 16 + l
                for j in range(D // 16):
                    sl = pl.ds(j * 16, 16)
                    rows_v[e, sl] = rows_v[e, sl] * ae
            return carry

        lax.fori_loop(0, K // 16, group, 0)
        pltpu.sync_copy(rows_v, acc.at[row_v], add=True)

    issue(0, bufs[0])
    issue(1, bufs[1])
    issue(2, bufs[2])

    def triple(t, carry):
        i = t * 3
        for u in range(3):
            drain(i + u, bufs[u])
            process(bufs[u])
            issue(i + u + 3, bufs[u])
        return carry

    lax.fori_loop(0, (NCHUNK - 5) // 3, triple, 0)
    for idx in range(NCHUNK - 5, NCHUNK):
        buf = bufs[idx % 3]
        drain(idx, buf)
        process(buf)
        if idx + 3 < NCHUNK:
            issue(idx + 3, buf)

    plsc.subcore_barrier()

    pltpu.sync_copy(acc.at[pl.ds(off, ROWS_PS)],
                    out_hbm.at[c, pl.ds(off, ROWS_PS)])

    @pl.when(s == 0)
    def _():
        pltpu.sync_copy(acc.at[pl.ds(ROWS_REM_OFF, ROWS_REM)],
                        out_hbm.at[c, pl.ds(ROWS_REM_OFF, ROWS_REM)])


def kernel(x, A_vals, relation_coeffs, gamma, beta, coeff_kernel, W, b,
           edge_row, edge_col, rel_values):
    edge_row = edge_row.astype(jnp.int32)
    edge_col = edge_col.astype(jnp.int32)
    rel_values = rel_values.astype(jnp.int32)
    av2 = A_vals.reshape(_GRID, _EB, D)
    rv2 = rel_values.reshape(_GRID, _EB, D)
    g2 = gamma.reshape(1, D)
    bt2 = beta.reshape(1, D)
    b2 = b.reshape(1, D)

    xbn, xb2, a2 = pl.pallas_call(
        _prep_body,
        grid=(_GRID,),
        in_specs=[
            pl.BlockSpec(memory_space=pltpu.SMEM),
            pl.BlockSpec((_NB, D), lambda i: (i, 0)),
            pl.BlockSpec((_NB, 1), lambda i: (i, 0)),
            pl.BlockSpec((1, D), lambda i: (0, 0)),
            pl.BlockSpec((1, D), lambda i: (0, 0)),
            pl.BlockSpec((1, _EB, D), lambda i: (i, 0, 0)),
            pl.BlockSpec((1, _EB, D), lambda i: (i, 0, 0)),
        ],
        out_specs=[
            pl.BlockSpec((_NB, D), lambda i: (i, 0)),
            pl.BlockSpec((_NB, D), lambda i: (i, 0)),
            pl.BlockSpec((1, _EB, D), lambda i: (i, 0, 0)),
        ],
        out_shape=[
            jax.ShapeDtypeStruct((N, D), jnp.float32),
            jax.ShapeDtypeStruct((N, D), jnp.float32),
            jax.ShapeDtypeStruct((_GRID, _EB, D), jnp.float32),
        ],
    )(relation_coeffs, x, coeff_kernel, g2, bt2, av2, rv2)

    a_flat = a2.reshape(E)
    col_chunks = edge_col.reshape(NW, NCHUNK, K)
    zeros = jnp.zeros((N, D), jnp.float32)
    partials = _sc_scatter(xbn, a_flat, edge_row, col_chunks, xb2, zeros)

    out = pl.pallas_call(
        _combine_body,
        grid=(_GRID,),
        in_specs=[
            pl.BlockSpec((1, _NB, D), lambda i: (0, i, 0)),
            pl.BlockSpec((1, _NB, D), lambda i: (1, i, 0)),
            pl.BlockSpec((D, D), lambda i: (0, 0)),
            pl.BlockSpec((1, D), lambda i: (0, 0)),
        ],
        out_specs=pl.BlockSpec((_NB, D), lambda i: (i, 0)),
        out_shape=jax.ShapeDtypeStruct((N, D), jnp.float32),
    )(partials, partials, W, b2)

    return out


# speedup vs baseline: 1.3454x; 1.0214x over previous
import functools

import jax
import jax.numpy as jnp
from jax import lax
from jax.experimental import pallas as pl
from jax.experimental.pallas import tpu as pltpu
from jax.experimental.pallas import tpu_sc as plsc

N = 10000
E = 320000
D = 128
R = 16
BN_EPS = 1e-3

NC = 2
NS = 16
NW = NC * NS
EW = E // NW
K = 80
NCHUNK = EW // K
ROWS_PS = 624
ROWS_REM_OFF = ROWS_PS * NS
ROWS_REM = N - ROWS_REM_OFF

_GRID = 10
_NB = N // _GRID
_EB = (E // D) // _GRID


def _prep_body(rc_ref, x_ref, ck_ref, g_ref, bt_ref, av_ref, rv_ref,
               xbn_ref, xb2_ref, a_ref):
    scale = g_ref[...] * (1.0 / jnp.sqrt(1.0 + BN_EPS))
    xbn = x_ref[...] * scale + bt_ref[...]
    xbn_ref[...] = xbn
    xb2_ref[...] = xbn * (ck_ref[...] + 1.0)
    rv = rv_ref[0]
    rel = jnp.zeros(rv.shape, jnp.float32)
    for r in range(R):
        rel = jnp.where(rv == r, rc_ref[r], rel)
    a_ref[0] = av_ref[0] / (1.0 + rel)


def _combine_body(p0_ref, p1_ref, w_ref, b_ref, o_ref):
    h = p0_ref[0] + p1_ref[0]
    o_ref[...] = jnp.dot(h, w_ref[...],
                         preferred_element_type=jnp.float32) + b_ref[...]


_sc_mesh = plsc.VectorSubcoreMesh(core_axis_name="c", subcore_axis_name="s")


@functools.partial(
    pl.kernel,
    out_type=jax.ShapeDtypeStruct((NC, N, D), jnp.float32),
    mesh=_sc_mesh,
    scratch_types=[
        pltpu.VMEM_SHARED((N, D), jnp.float32),
        pltpu.VMEM((NCHUNK, K), jnp.int32),
        pltpu.VMEM((48,), jnp.int32),
        pltpu.VMEM((48,), jnp.int32),
        pltpu.VMEM((48,), jnp.int32),
        pltpu.VMEM((32,), jnp.int32),
        pltpu.VMEM((32,), jnp.int32),
        pltpu.VMEM((32,), jnp.int32),
        pltpu.VMEM((K,), jnp.float32),
        pltpu.VMEM((K,), jnp.float32),
        pltpu.VMEM((K,), jnp.float32),
        pltpu.VMEM((K, D), jnp.float32),
        pltpu.VMEM((K, D), jnp.float32),
        pltpu.VMEM((K, D), jnp.float32),
        pltpu.SemaphoreType.DMA,
        pltpu.SemaphoreType.DMA,
        pltpu.SemaphoreType.DMA,
        pltpu.SemaphoreType.DMA,
    ],
)
def _sc_scatter(xbn_hbm, a_hbm, row_hbm, col_hbm, init_hbm, zero_hbm,
                out_hbm, acc, col2, rowa0, rowa1, rowa2, rowc0, rowc1, rowc2,
                ab0, ab1, ab2, rows0, rows1, rows2, sem0, sem1, sem2, sems):
    c = lax.axis_index("c")
    s = lax.axis_index("s")
    w = c * NS + s
    off = s * ROWS_PS

    @pl.when(c == 0)
    def _():
        pltpu.sync_copy(init_hbm.at[pl.ds(off, ROWS_PS)],
                        acc.at[pl.ds(off, ROWS_PS)])

    @pl.when(c != 0)
    def _():
        pltpu.sync_copy(zero_hbm.at[pl.ds(off, ROWS_PS)],
                        acc.at[pl.ds(off, ROWS_PS)])

    @pl.when((s == 0) & (c == 0))
    def _():
        pltpu.sync_copy(init_hbm.at[pl.ds(ROWS_REM_OFF, ROWS_REM)],
                        acc.at[pl.ds(ROWS_REM_OFF, ROWS_REM)])

    @pl.when((s == 0) & (c != 0))
    def _():
        pltpu.sync_copy(zero_hbm.at[pl.ds(ROWS_REM_OFF, ROWS_REM)],
                        acc.at[pl.ds(ROWS_REM_OFF, ROWS_REM)])

    pltpu.sync_copy(col_hbm.at[w], col2)

    plsc.subcore_barrier()

    ebase = w * EW
    bufs = ((rows0, rowa0, rowc0, ab0, sem0),
            (rows1, rowa1, rowc1, ab1, sem1),
            (rows2, rowa2, rowc2, ab2, sem2))

    def issue(i, buf):
        rows_v, rowa_v, rowc_v, a_v, sem = buf
        pltpu.async_copy(xbn_hbm.at[col2.at[i]], rows_v, sem)
        pltpu.async_copy(row_hbm.at[pl.ds(ebase + i * K, 48)], rowa_v, sem)
        pltpu.async_copy(row_hbm.at[pl.ds(ebase + i * K + 48, 32)],
                         rowc_v, sem)
        pltpu.async_copy(a_hbm.at[pl.ds(ebase + i * K, K)], a_v, sem)

    def drain(i, buf):
        rows_v, rowa_v, rowc_v, a_v, sem = buf
        pltpu.make_async_copy(xbn_hbm.at[col2.at[i]], rows_v, sem).wait()
        pltpu.make_async_copy(row_hbm.at[pl.ds(ebase + i * K, 48)],
                              rowa_v, sem).wait()
        pltpu.make_async_copy(row_hbm.at[pl.ds(ebase + i * K + 48, 32)],
                              rowc_v, sem).wait()
        pltpu.make_async_copy(a_hbm.at[pl.ds(ebase + i * K, K)],
                              a_v, sem).wait()

    def process(buf):
        rows_v, rowa_v, rowc_v, a_v, _ = buf

        def group(g, carry):
            a16 = a_v[pl.ds(g * 16, 16)]
            for l in range(16):
                ae = a16[l]
                e = g * 16 + l
                for j in range(D // 16):
                    sl = pl.ds(j * 16, 16)
                    rows_v[e, sl] = rows_v[e, sl] * ae
            return carry

        lax.fori_loop(0, 3, group, 0)
        pltpu.async_copy(rows_v.at[pl.ds(0, 48)], acc.at[rowa_v], sems,
                         add=True)
        lax.fori_loop(3, K // 16, group, 0)
        pltpu.sync_copy(rows_v.at[pl.ds(48, 32)], acc.at[rowc_v], add=True)
        pltpu.make_async_copy(rows_v.at[pl.ds(0, 48)], acc.at[rowa_v],
                              sems).wait()

    issue(0, bufs[0])
    issue(1, bufs[1])
    issue(2, bufs[2])

    def triple(t, carry):
        i = t * 3
        for u in range(3):
            drain(i + u, bufs[u])
            process(bufs[u])
            issue(i + u + 3, bufs[u])
        return carry

    lax.fori_loop(0, (NCHUNK - 5) // 3, triple, 0)
    for idx in range(NCHUNK - 5, NCHUNK):
        buf = bufs[idx % 3]
        drain(idx, buf)
        process(buf)
        if idx + 3 < NCHUNK:
            issue(idx + 3, buf)

    plsc.subcore_barrier()

    pltpu.sync_copy(acc.at[pl.ds(off, ROWS_PS)],
                    out_hbm.at[c, pl.ds(off, ROWS_PS)])

    @pl.when(s == 0)
    def _():
        pltpu.sync_copy(acc.at[pl.ds(ROWS_REM_OFF, ROWS_REM)],
                        out_hbm.at[c, pl.ds(ROWS_REM_OFF, ROWS_REM)])


def kernel(x, A_vals, relation_coeffs, gamma, beta, coeff_kernel, W, b,
           edge_row, edge_col, rel_values):
    edge_row = edge_row.astype(jnp.int32)
    edge_col = edge_col.astype(jnp.int32)
    rel_values = rel_values.astype(jnp.int32)
    av2 = A_vals.reshape(_GRID, _EB, D)
    rv2 = rel_values.reshape(_GRID, _EB, D)
    g2 = gamma.reshape(1, D)
    bt2 = beta.reshape(1, D)
    b2 = b.reshape(1, D)

    xbn, xb2, a2 = pl.pallas_call(
        _prep_body,
        grid=(_GRID,),
        in_specs=[
            pl.BlockSpec(memory_space=pltpu.SMEM),
            pl.BlockSpec((_NB, D), lambda i: (i, 0)),
            pl.BlockSpec((_NB, 1), lambda i: (i, 0)),
            pl.BlockSpec((1, D), lambda i: (0, 0)),
            pl.BlockSpec((1, D), lambda i: (0, 0)),
            pl.BlockSpec((1, _EB, D), lambda i: (i, 0, 0)),
            pl.BlockSpec((1, _EB, D), lambda i: (i, 0, 0)),
        ],
        out_specs=[
            pl.BlockSpec((_NB, D), lambda i: (i, 0)),
            pl.BlockSpec((_NB, D), lambda i: (i, 0)),
            pl.BlockSpec((1, _EB, D), lambda i: (i, 0, 0)),
        ],
        out_shape=[
            jax.ShapeDtypeStruct((N, D), jnp.float32),
            jax.ShapeDtypeStruct((N, D), jnp.float32),
            jax.ShapeDtypeStruct((_GRID, _EB, D), jnp.float32),
        ],
    )(relation_coeffs, x, coeff_kernel, g2, bt2, av2, rv2)

    a_flat = a2.reshape(E)
    col_chunks = edge_col.reshape(NW, NCHUNK, K)
    zeros = jnp.zeros((N, D), jnp.float32)
    partials = _sc_scatter(xbn, a_flat, edge_row, col_chunks, xb2, zeros)

    out = pl.pallas_call(
        _combine_body,
        grid=(_GRID,),
        in_specs=[
            pl.BlockSpec((1, _NB, D), lambda i: (0, i, 0)),
            pl.BlockSpec((1, _NB, D), lambda i: (1, i, 0)),
            pl.BlockSpec((D, D), lambda i: (0, 0)),
            pl.BlockSpec((1, D), lambda i: (0, 0)),
        ],
        out_specs=pl.BlockSpec((_NB, D), lambda i: (i, 0)),
        out_shape=jax.ShapeDtypeStruct((N, D), jnp.float32),
    )(partials, partials, W, b2)

    return out
